# baseline (speedup 1.0000x reference)
import functools

import jax
import jax.numpy as jnp
import numpy as np
from jax import lax
from jax.experimental import pallas as pl
from jax.experimental.pallas import tpu as pltpu
from jax.experimental.pallas import tpu_sc as plsc

NUM_OBJECTS = 8
NUM_PARTS = 64
L_ENC = 10
E = NUM_OBJECTS + 1
TILE = 1024
NW = 32
LANES = 16
ROW = 128


def _obj_affine():
    rs = np.random.RandomState(0)
    thetas = rs.uniform(-np.pi, np.pi, size=(NUM_OBJECTS,))
    rots = []
    for th in thetas:
        c, s = np.cos(th), np.sin(th)
        m = np.array([[c, -s, 0.0], [s, c, 0.0], [0.0, 0.0, 1.0]], dtype=np.float64)
        rots.append(np.linalg.inv(m))
    rots.append(np.zeros((3, 3)))
    rot = np.stack(rots, 0).astype(np.float32)
    loc = rs.uniform(-20, 20, size=(NUM_OBJECTS, 3))
    loc = np.concatenate([loc, np.zeros((1, 3))], 0).astype(np.float32)
    half = rs.uniform(0.5, 5.0, size=(NUM_OBJECTS, 3))
    dim = np.stack([-half, half], 1)
    dim = np.concatenate([dim, np.zeros((1, 2, 3))], 0).astype(np.float32)
    scale = np.amax(dim[:, 1, :] - dim[:, 0, :], -1)
    scale[-1] = 1.0
    center = (dim[:, 1, :] + dim[:, 0, :]) / 2.0
    return np.concatenate(
        [rot.reshape(E, 9), loc, center, scale[:, None]], axis=1
    ).astype(np.float32)


_PARAMS = _obj_affine()

_TWOPI = 2.0 * np.pi
_INV2PI = float(np.float32(1.0 / _TWOPI))


def _split9(v):
    f = np.float32(v)
    bits = f.view(np.uint32) & np.uint32(0xFFFF8000)
    return float(bits.view(np.float32))


_TWOPI_1 = _split9(_TWOPI)
_TWOPI_2 = _split9(_TWOPI - _TWOPI_1)
_TWOPI_3 = float(np.float32(_TWOPI - _TWOPI_1 - _TWOPI_2))

_ENC_PERM = np.array(
    [3 + 6 * (f // 3) + f % 3 for f in range(30)]
    + [6 + 6 * (f // 3) + f % 3 for f in range(30)]
    + [0, 1, 2], dtype=np.int32)


def _expert_scalar(x):
    return jnp.where((x == -1) | (x == NUM_PARTS), NUM_OBJECTS,
                     x % NUM_OBJECTS)


def _wid():
    return lax.axis_index("s") * 2 + lax.axis_index("c")


def _sc_hist_kernel(CH, s_hbm, cnt_hbm, s_v, cntv_v, cnt_sm):
    w = _wid()
    pltpu.sync_copy(s_hbm.at[w], s_v.at[pl.ds(0, CH)])
    for e in range(LANES):
        cnt_sm[e] = 0

    def body(i, carry):
        ev = s_v[pl.ds(i, LANES)]
        e = _expert_scalar(ev[0])
        cnt_sm[e] = cnt_sm[e] + 1
        return carry

    lax.fori_loop(0, CH, body, 0)
    lane = lax.iota(jnp.int32, LANES)
    acc = jnp.zeros((LANES,), jnp.int32)
    for e in range(LANES):
        acc = jnp.where(lane == e, cnt_sm[e], acc)
    cntv_v[pl.ds(0, LANES)] = acc
    pltpu.sync_copy(cntv_v, cnt_hbm.at[w])


def _sc_disp_kernel(CH, s_hbm, cnt_hbm, x_hbm, xs_hbm, dst_hbm, bins_hbm,
                    s_v, cnt_v, dst2_v, x_v, bins_v, base_sm, dst_sm, sem):
    w = _wid()
    pltpu.sync_copy(s_hbm.at[w], s_v.at[pl.ds(0, CH)])
    pltpu.sync_copy(cnt_hbm, cnt_v)
    pltpu.sync_copy(x_hbm.at[pl.ds(w * CH, CH)], x_v)
    total = jnp.zeros((LANES,), jnp.int32)
    mine = jnp.zeros((LANES,), jnp.int32)
    for i in range(NW):
        row = cnt_v[i]
        total = total + row
        mine = mine + row * (jnp.int32(i) < w).astype(jnp.int32)
    lane = lax.iota(jnp.int32, LANES)
    run = jnp.int32(0)
    binsacc = jnp.zeros((LANES,), jnp.int32)
    for e in range(E):
        binsacc = jnp.where(lane == e, run, binsacc)
        base_sm[e] = run + mine[e]
        run = run + total[e]
    for e in range(E, LANES):
        binsacc = jnp.where(lane == e, run, binsacc)
    bins_v[pl.ds(0, LANES)] = binsacc

    @pl.when(w == 0)
    def _():
        pltpu.sync_copy(bins_v, bins_hbm)

    def body(i, carry):
        ev = s_v[pl.ds(i, LANES)]
        e = _expert_scalar(ev[0])
        d = base_sm[e]
        base_sm[e] = d + 1
        dst_sm[i] = d
        return carry

    lax.fori_loop(0, CH, body, 0)
    for v in range(CH // LANES):
        acc = jnp.zeros((LANES,), jnp.int32)
        for l in range(LANES):
            acc = jnp.where(lane == l, dst_sm[v * LANES + l], acc)
        dst2_v[v // 8, pl.ds((v % 8) * LANES, LANES)] = acc
    pltpu.sync_copy(dst2_v, dst_hbm.at[w])
    for j in range(CH // 128):
        pltpu.async_copy(x_v.at[pl.ds(j * 128, 128)],
                         xs_hbm.at[dst2_v.at[j]], sem).wait()


def _sc_comb_kernel(CH, res_hbm, dst_hbm, out_hbm, dst2_v, rows_v, sem):
    w = _wid()
    pltpu.sync_copy(dst_hbm.at[w], dst2_v)
    for j in range(CH // 128):
        pltpu.async_copy(res_hbm.at[dst2_v.at[j]],
                         rows_v.at[pl.ds(j * 128, 128)], sem).wait()
    pltpu.sync_copy(rows_v, out_hbm.at[pl.ds(w * CH, CH)])


def _mlp_kernel(bins_ref, x_ref, params_ref,
                w0_ref, b0_ref, w1_ref, b1_ref, w2_ref, b2_ref,
                iw0_ref, ib0_ref, iw1_ref, ib1_ref, iw2_ref, ib2_ref,
                iw3_ref, ib3_ref, iw4_ref, ib4_ref, out_ref, z_sc):
    f32 = jnp.float32
    dot = functools.partial(jnp.dot, preferred_element_type=f32)
    dot_hi = functools.partial(jnp.dot, preferred_element_type=f32,
                               precision=jax.lax.Precision.HIGHEST)

    row0 = pl.program_id(0) * TILE
    rows = row0 + jax.lax.broadcasted_iota(jnp.int32, (TILE, 1), 0)
    oi = jnp.zeros((TILE, 1), jnp.int32)
    for e in range(1, E):
        oi = oi + (rows >= bins_ref[e]).astype(jnp.int32)

    eids = jax.lax.broadcasted_iota(jnp.int32, (TILE, E), 1)
    onehot = (oi == eids).astype(f32)
    pp = dot_hi(onehot, params_ref[:, :])

    d = [x_ref[:, j:j + 1] - pp[:, 9 + j:10 + j] for j in range(3)]
    xu = []
    for j in range(3):
        m = pp[:, 3 * j:3 * j + 1] * d[0] + pp[:, 3 * j + 1:3 * j + 2] * d[1] \
            + pp[:, 3 * j + 2:3 * j + 3] * d[2]
        xu.append((m - pp[:, 12 + j:13 + j]) / pp[:, 15:16] * 2.0)
    xu = jnp.concatenate(xu, axis=-1)

    inv2pi = jnp.float32(_INV2PI)
    c1, c2, c3 = (jnp.float32(_TWOPI_1), jnp.float32(_TWOPI_2),
                  jnp.float32(_TWOPI_3))
    scm = jnp.concatenate([xu * (2.0 ** i) for i in range(L_ENC)], axis=-1)
    k = jnp.floor(scm * inv2pi + 0.5)
    r = ((scm - k * c1) - k * c2) - k * c3
    argm = jnp.concatenate([r, r + jnp.float32(np.pi / 2),
                            jnp.zeros((TILE, 4), f32)], axis=-1)
    sins = jnp.sin(argm)
    enc = jnp.concatenate([sins[:, :60], xu,
                           jnp.zeros((TILE, 1), f32)], axis=-1)

    geo = x_ref[:, 16:32]
    z_sc[:, :] = jnp.concatenate([jnp.zeros((TILE, 48), f32), geo], axis=-1)
    for e in range(E):
        lo = bins_ref[e]
        hi = bins_ref[e + 1]

        @pl.when((lo < row0 + TILE) & (hi > row0))
        def _(e=e, lo=lo, hi=hi):
            h = jnp.maximum(dot(enc, w0_ref[e]) + b0_ref[e], 0.0)
            h = jnp.maximum(dot(h, w1_ref[e]) + b1_ref[e], 0.0)
            ze = dot(h, w2_ref[e]) + b2_ref[e]
            m = ((rows >= lo) & (rows < hi)).astype(f32)
            z_sc[:, :] = z_sc[:, :] + m * ze

    z = z_sc[:, :]
    a = jnp.maximum(dot(z, iw0_ref[:, :]) + ib0_ref[:, :], 0.0)
    a = jnp.maximum(dot(a, iw1_ref[:, :]) + ib1_ref[:, :], 0.0)
    a = jnp.maximum(dot(a, iw2_ref[:, :]) + ib2_ref[:, :], 0.0)
    a = jnp.maximum(dot(a, iw3_ref[:, :]) + ib3_ref[:, :], 0.0)
    out_ref[:, :] = dot(a, iw4_ref[:, :]) + ib4_ref[:, :]


def kernel(sionna_obj_idx, pos, geo_feat, interaction_type,
           objW0, objb0, objW1, objb1, objW2, objb2,
           intW0, intb0, intW1, intb1, intW2, intb2, intW3, intb3,
           intW4, intb4):
    data_shape = sionna_obj_idx.shape
    N = int(np.prod(data_shape))
    NT = N // TILE
    CH = N // NW
    f32 = jnp.float32
    i32 = jnp.int32

    s_w = sionna_obj_idx.reshape(NW, CH).astype(i32)
    X = jnp.concatenate(
        [pos.reshape(N, 3).astype(f32),
         jnp.zeros((N, 13), f32),
         geo_feat.reshape(N, 16).astype(f32),
         jnp.zeros((N, ROW - 32), f32)], axis=-1)

    mesh = plsc.VectorSubcoreMesh(core_axis_name="c", subcore_axis_name="s")

    (cnt_w,) = pl.kernel(
        functools.partial(_sc_hist_kernel, CH), mesh=mesh,
        out_type=[jax.ShapeDtypeStruct((NW, LANES), i32)],
        scratch_types=[pltpu.VMEM((CH + LANES,), i32),
                       pltpu.VMEM((LANES,), i32),
                       pltpu.SMEM((LANES,), i32)],
    )(s_w)

    xs, dst_w, bins = pl.kernel(
        functools.partial(_sc_disp_kernel, CH), mesh=mesh,
        out_type=[jax.ShapeDtypeStruct((N, ROW), f32),
                  jax.ShapeDtypeStruct((NW, CH // 128, 128), i32),
                  jax.ShapeDtypeStruct((LANES,), i32)],
        scratch_types=[pltpu.VMEM((CH + LANES,), i32),
                       pltpu.VMEM((NW, LANES), i32),
                       pltpu.VMEM((CH // 128, 128), i32),
                       pltpu.VMEM((CH, ROW), f32),
                       pltpu.VMEM((LANES,), i32),
                       pltpu.SMEM((LANES,), i32),
                       pltpu.SMEM((CH,), i32),
                       pltpu.SemaphoreType.DMA],
    )(s_w, cnt_w, X)

    params = jnp.asarray(_PARAMS)
    w0p = jnp.concatenate([objW0[:, _ENC_PERM, :],
                           jnp.zeros((E, 1, 64), f32)], axis=1)
    w2p = jnp.concatenate([objW2, jnp.zeros((E, 64, 16), f32)], axis=2)
    b2p = jnp.concatenate([objb2, jnp.zeros((E, 16), f32)], axis=1)
    w4p = jnp.concatenate([intW4, jnp.zeros((128, ROW - 8), f32)], axis=1)
    b4p = jnp.concatenate([intb4, jnp.zeros((ROW - 8,), f32)]).reshape(1, ROW)

    full = lambda shape: pl.BlockSpec(shape, lambda i, b: (0,) * len(shape))
    grid_spec = pltpu.PrefetchScalarGridSpec(
        num_scalar_prefetch=1,
        grid=(NT,),
        in_specs=[
            pl.BlockSpec((TILE, ROW), lambda i, b: (i, 0)),
            full((E, 16)),
            full((E, 64, 64)), full((E, 64)),
            full((E, 64, 64)), full((E, 64)),
            full((E, 64, 64)), full((E, 64)),
            full((64, 128)), full((1, 128)),
            full((128, 128)), full((1, 128)),
            full((128, 128)), full((1, 128)),
            full((128, 128)), full((1, 128)),
            full((128, ROW)), full((1, ROW)),
        ],
        out_specs=pl.BlockSpec((TILE, ROW), lambda i, b: (i, 0)),
        scratch_shapes=[pltpu.VMEM((TILE, 64), f32)],
    )
    res = pl.pallas_call(
        _mlp_kernel,
        grid_spec=grid_spec,
        out_shape=jax.ShapeDtypeStruct((N, ROW), f32),
    )(bins, xs, params,
      w0p, objb0, objW1, objb1, w2p, b2p,
      intW0, intb0.reshape(1, 128), intW1, intb1.reshape(1, 128),
      intW2, intb2.reshape(1, 128), intW3, intb3.reshape(1, 128),
      w4p, b4p)

    (outg,) = pl.kernel(
        functools.partial(_sc_comb_kernel, CH), mesh=mesh,
        out_type=[jax.ShapeDtypeStruct((N, ROW), f32)],
        scratch_types=[pltpu.VMEM((CH // 128, 128), i32),
                       pltpu.VMEM((CH, ROW), f32),
                       pltpu.SemaphoreType.DMA],
    )(res, dst_w)

    o = outg[:, :8]
    tc = jax.lax.complex(o[:, :4], o[:, 4:8]).reshape(data_shape + (4,))
    return jnp.stack(jnp.split(tc, 2, axis=-1), -1)
